# baseline (speedup 1.0000x reference)
import functools

import jax
import jax.numpy as jnp
from jax import lax
from jax.experimental import pallas as pl
from jax.experimental.pallas import tpu as pltpu
from jax.experimental.pallas import tpu_sc as plsc


def kernel(scores, boxes):
    del boxes
    n = scores.shape[0]
    chunk = 200
    num_chunks = n // chunk
    assert num_chunks * chunk == n

    info = plsc.get_sparse_core_info()
    nc = info.num_cores
    mesh = plsc.VectorSubcoreMesh(core_axis_name="c", subcore_axis_name="s")

    @functools.partial(
        pl.kernel,
        mesh=mesh,
        out_type=jax.ShapeDtypeStruct((n,), scores.dtype),
        scratch_types=[pltpu.VMEM((chunk,), scores.dtype)],
    )
    def copy_k(scores_hbm, out_hbm, buf):
        wid = lax.axis_index("s") * nc + lax.axis_index("c")

        @pl.when(wid < num_chunks)
        def _():
            base = wid * chunk
            pltpu.sync_copy(scores_hbm.at[pl.ds(base, chunk)], buf)
            pltpu.sync_copy(buf, out_hbm.at[pl.ds(base, chunk)])

    return copy_k(scores)
